# baseline (speedup 1.0000x reference)
import functools

import jax
import jax.numpy as jnp
from jax import lax
from jax.experimental import pallas as pl
from jax.experimental.pallas import tpu as pltpu
from jax.experimental.pallas import tpu_sc as plsc

_NUM_CORES = 2
_NUM_SUBCORES = 16
_NUM_WORKERS = _NUM_CORES * _NUM_SUBCORES
_GW = 128
_CHUNK_ROWS = 4


@functools.lru_cache(maxsize=None)
def _make_lookup(n_rows: int, dim: int):
    rows_per_w = n_rows // _NUM_WORKERS
    n_chunks = rows_per_w // _CHUNK_ROWS
    mesh = plsc.VectorSubcoreMesh(core_axis_name="c", subcore_axis_name="s")

    @functools.partial(
        pl.kernel,
        out_type=jax.ShapeDtypeStruct((n_rows * _GW, dim), jnp.float32),
        mesh=mesh,
        scratch_types=[
            pltpu.VMEM((_CHUNK_ROWS, _GW), jnp.int32),
            pltpu.VMEM((_CHUNK_ROWS * _GW, dim), jnp.float32),
            pltpu.SemaphoreType.DMA,
        ],
        compiler_params=pltpu.CompilerParams(use_tc_tiling_on_sc=False),
    )
    def lookup(idx_hbm, table_hbm, out_hbm, idx_v, rows_v, gsem):
        wid = lax.axis_index("s") * _NUM_CORES + lax.axis_index("c")
        row_base = wid * rows_per_w

        def chunk(c, carry):
            r0 = row_base + c * _CHUNK_ROWS
            pltpu.sync_copy(idx_hbm.at[pl.ds(r0, _CHUNK_ROWS)], idx_v)
            copies = [
                pltpu.async_copy(
                    table_hbm.at[idx_v.at[j]],
                    rows_v.at[pl.ds(j * _GW, _GW)],
                    gsem,
                )
                for j in range(_CHUNK_ROWS)
            ]
            for cp in copies:
                cp.wait()
            pltpu.sync_copy(rows_v, out_hbm.at[pl.ds(r0 * _GW, _CHUNK_ROWS * _GW)])
            return carry

        lax.fori_loop(0, n_chunks, chunk, None)

    return lookup


def kernel(token_ids, weight):
    b, s = token_ids.shape
    n = b * s
    dim = weight.shape[1]
    idx2d = token_ids.reshape(n // _GW, _GW).astype(jnp.int32)
    flat = _make_lookup(n // _GW, dim)(idx2d, weight)
    return flat.reshape(b, s, dim)


# speedup vs baseline: 1.0425x; 1.0425x over previous
import functools

import jax
import jax.numpy as jnp
from jax import lax
from jax.experimental import pallas as pl
from jax.experimental.pallas import tpu as pltpu
from jax.experimental.pallas import tpu_sc as plsc

_NUM_CORES = 2
_NUM_SUBCORES = 16
_NUM_WORKERS = _NUM_CORES * _NUM_SUBCORES
_GW = 128
_CHUNK_ROWS = 5
_NBUF = 2


@functools.lru_cache(maxsize=None)
def _make_lookup(n_rows: int, dim: int):
    rows_per_w = n_rows // _NUM_WORKERS
    n_chunks = rows_per_w // _CHUNK_ROWS
    n_groups = n_chunks // _NBUF
    assert rows_per_w == n_chunks * _CHUNK_ROWS and n_chunks == n_groups * _NBUF
    mesh = plsc.VectorSubcoreMesh(core_axis_name="c", subcore_axis_name="s")

    @functools.partial(
        pl.kernel,
        out_type=jax.ShapeDtypeStruct((n_rows * _GW, dim), jnp.float32),
        mesh=mesh,
        scratch_types=[
            pltpu.VMEM((_NBUF, _CHUNK_ROWS, _GW), jnp.int32),
            pltpu.VMEM((_NBUF, _CHUNK_ROWS * _GW, dim), jnp.float32),
        ]
        + [pltpu.SemaphoreType.DMA] * (3 * _NBUF),
        compiler_params=pltpu.CompilerParams(use_tc_tiling_on_sc=False),
    )
    def lookup(idx_hbm, table_hbm, out_hbm, idx_v, rows_v, *sems):
        isem = sems[:_NBUF]
        gsem = sems[_NBUF:2 * _NBUF]
        osem = sems[2 * _NBUF:]
        wid = lax.axis_index("s") * _NUM_CORES + lax.axis_index("c")
        row_base = wid * rows_per_w

        def fetch_idx(c, b):
            return pltpu.async_copy(
                idx_hbm.at[pl.ds(row_base + c * _CHUNK_ROWS, _CHUNK_ROWS)],
                idx_v.at[b], isem[b])

        def run_gathers(b):
            copies = [
                pltpu.async_copy(
                    table_hbm.at[idx_v.at[b, j]],
                    rows_v.at[b, pl.ds(j * _GW, _GW)],
                    gsem[b],
                )
                for j in range(_CHUNK_ROWS)
            ]
            for cp in copies:
                cp.wait()

        def put_out(c, b):
            return pltpu.async_copy(
                rows_v.at[b],
                out_hbm.at[pl.ds((row_base + c * _CHUNK_ROWS) * _GW,
                                 _CHUNK_ROWS * _GW)],
                osem[b])

        def drain_out(b):
            pltpu.make_async_copy(
                rows_v.at[b],
                out_hbm.at[pl.ds(row_base * _GW, _CHUNK_ROWS * _GW)],
                osem[b]).wait()

        idx_cp = [fetch_idx(b, b) for b in range(_NBUF)]
        for b in range(_NBUF):
            idx_cp[b].wait()
            run_gathers(b)
            fetch_idx(b + _NBUF, b)
            put_out(b, b)

        def group(g, carry):
            for b in range(_NBUF):
                c = g * _NBUF + b
                pltpu.make_async_copy(
                    idx_hbm.at[pl.ds(row_base, _CHUNK_ROWS)],
                    idx_v.at[b], isem[b]).wait()
                drain_out(b)
                run_gathers(b)
                fetch_idx(lax.min(c + _NBUF, n_chunks - 1), b)
                put_out(c, b)
            return carry

        lax.fori_loop(1, n_groups, group, None)

        for b in range(_NBUF):
            pltpu.make_async_copy(
                idx_hbm.at[pl.ds(row_base, _CHUNK_ROWS)],
                idx_v.at[b], isem[b]).wait()
            drain_out(b)

    return lookup


def kernel(token_ids, weight):
    b, s = token_ids.shape
    n = b * s
    dim = weight.shape[1]
    idx2d = token_ids.reshape(n // _GW, _GW).astype(jnp.int32)
    flat = _make_lookup(n // _GW, dim)(idx2d, weight)
    return flat.reshape(b, s, dim)
